# baseline (speedup 1.0000x reference)
import jax
import jax.numpy as jnp
from jax import lax
from jax.experimental import pallas as pl
from jax.experimental.pallas import tpu as pltpu
from jax.experimental.pallas import tpu_sc as plsc

_B = 16384
_D = 32
_NW = 32
_BPW = _B // _NW
_CH = 128
_NCHUNK = _BPW // _CH
_HALF = _BPW // 2
_ROWW = 128
_L = 16


def _pmf_body(q_u_hbm, q_i_hbm, p_u_hbm, p_i_hbm, uemb, iemb, out_hbm,
              idx_u, idx_i, pe_u, pe_i, rows_u, rows_i, out_v, sem_u, sem_i):
    wid = lax.axis_index("s") * 2 + lax.axis_index("c")

    pltpu.sync_copy(q_u_hbm.at[wid], idx_u)
    pltpu.sync_copy(q_i_hbm.at[wid], idx_i)
    pltpu.sync_copy(p_u_hbm.at[wid], pe_u)
    pltpu.sync_copy(p_i_hbm.at[wid], pe_i)

    lane = lax.iota(jnp.int32, _L)
    lane_row = lane * _ROWW

    def half_body(h, carry):
        copies = []
        for j in range(2):
            sl = pl.ds(j * _CH, _CH)
            copies.append(pltpu.async_copy(
                uemb.at[idx_u.at[2 * h + j]], rows_u.at[sl], sem_u))
            copies.append(pltpu.async_copy(
                iemb.at[idx_i.at[2 * h + j]], rows_i.at[sl], sem_i))
        for c in copies:
            c.wait()

        def group_body(g, carry2):
            e0 = g * _L
            m0 = h * _HALF + e0
            row_u = lax.broadcast_in_dim(e0, (_L,), ()) + lane
            base_u = lane_row + pe_u[pl.ds(m0, _L)]
            base_i = lane_row + pe_i[pl.ds(m0, _L)]
            acc = jnp.zeros((_L,), jnp.float32)
            for d in range(_D):
                uv = plsc.load_gather(rows_u, [row_u, base_u + d - lane_row])
                iv = plsc.load_gather(rows_i, [row_u, base_i + d - lane_row])
                acc = acc + uv * iv
            out_v[pl.ds(m0, _L)] = acc
            return carry2

        lax.fori_loop(0, _HALF // _L, group_body, 0)
        return carry

    lax.fori_loop(0, 2, half_body, 0)

    pltpu.sync_copy(out_v, out_hbm.at[pl.ds(wid * _BPW, _BPW)])


@jax.jit
def _pmf(q_u, q_i, p_u, p_i, uemb4, iemb4):
    mesh = plsc.VectorSubcoreMesh(core_axis_name="c", subcore_axis_name="s")
    return pl.kernel(
        _pmf_body,
        out_type=jax.ShapeDtypeStruct((_B,), jnp.float32),
        mesh=mesh,
        compiler_params=pltpu.CompilerParams(
            needs_layout_passes=False, use_tc_tiling_on_sc=True),
        scratch_types=[
            pltpu.VMEM((_NCHUNK, _CH), jnp.int32),
            pltpu.VMEM((_NCHUNK, _CH), jnp.int32),
            pltpu.VMEM((_BPW,), jnp.int32),
            pltpu.VMEM((_BPW,), jnp.int32),
            pltpu.VMEM((_HALF, _ROWW), jnp.float32),
            pltpu.VMEM((_HALF, _ROWW), jnp.float32),
            pltpu.VMEM((_BPW,), jnp.float32),
            pltpu.SemaphoreType.DMA,
            pltpu.SemaphoreType.DMA,
        ],
    )(q_u, q_i, p_u, p_i, uemb4, iemb4)


def kernel(u, i, user_emb, item_emb):
    u32 = u.astype(jnp.int32)
    i32 = i.astype(jnp.int32)
    q_u = (u32 // 4).reshape(_NW, _NCHUNK, _CH)
    q_i = (i32 // 4).reshape(_NW, _NCHUNK, _CH)
    p_u = ((u32 % 4) * _D).reshape(_NW, _BPW)
    p_i = ((i32 % 4) * _D).reshape(_NW, _BPW)
    uemb4 = user_emb.reshape(250000, _ROWW)
    iemb4 = item_emb.reshape(250000, _ROWW)
    return _pmf(q_u, q_i, p_u, p_i, uemb4, iemb4)


# speedup vs baseline: 1.3760x; 1.3760x over previous
import jax
import jax.numpy as jnp
from jax import lax
from jax.experimental import pallas as pl
from jax.experimental.pallas import tpu as pltpu
from jax.experimental.pallas import tpu_sc as plsc

_B = 16384
_D = 32
_NW = 32
_BPW = _B // _NW
_C = 32
_NCHUNK = _BPW // _C
_L = 16


def _pmf_body(u_hbm, i_hbm, uemb, iemb, out_hbm,
              u_v, i_v, blk_u, blk_i, out_v, sem_u, sem_i):
    wid = lax.axis_index("s") * 2 + lax.axis_index("c")

    pltpu.sync_copy(u_hbm.at[wid], u_v)
    pltpu.sync_copy(i_hbm.at[wid], i_v)

    lane = lax.iota(jnp.int32, _L)
    lane8 = lane * 8

    def chunk_body(c, carry):
        c0 = c * _C
        qvs = []
        copies = []
        for g in range(_C // _L):
            qu = u_v[pl.ds(c0 + g * _L, _L)]
            qi = i_v[pl.ds(c0 + g * _L, _L)]
            qvs.append((qu, qi))
            for k in range(_L):
                s = g * _L + k
                cu = jnp.sum(jnp.where(lane == k, qu, 0))
                copies.append(pltpu.async_copy(
                    uemb.at[pl.ds(pl.multiple_of((cu >> 3) * 8, 8), 8), :],
                    blk_u.at[pl.ds(s * 8, 8), :], sem_u))
                ci = jnp.sum(jnp.where(lane == k, qi, 0))
                copies.append(pltpu.async_copy(
                    iemb.at[pl.ds(pl.multiple_of((ci >> 3) * 8, 8), 8), :],
                    blk_i.at[pl.ds(s * 8, 8), :], sem_i))
        for cp in copies:
            cp.wait()

        for g in range(_C // _L):
            qu, qi = qvs[g]
            row_u = lane8 + (qu & 7) + g * (_L * 8)
            row_i = lane8 + (qi & 7) + g * (_L * 8)
            acc = jnp.zeros((_L,), jnp.float32)
            for d in range(_D):
                dv = jnp.full((_L,), d, jnp.int32)
                uv = plsc.load_gather(blk_u, [row_u, dv])
                iv = plsc.load_gather(blk_i, [row_i, dv])
                acc = acc + uv * iv
            out_v[pl.ds(c0 + g * _L, _L)] = acc
        return carry

    lax.fori_loop(0, _NCHUNK, chunk_body, 0)

    pltpu.sync_copy(out_v, out_hbm.at[pl.ds(wid * _BPW, _BPW)])


@jax.jit
def _pmf(u2, i2, user_emb, item_emb):
    mesh = plsc.VectorSubcoreMesh(core_axis_name="c", subcore_axis_name="s")
    return pl.kernel(
        _pmf_body,
        out_type=jax.ShapeDtypeStruct((_B,), jnp.float32),
        mesh=mesh,
        compiler_params=pltpu.CompilerParams(
            needs_layout_passes=False, use_tc_tiling_on_sc=True),
        scratch_types=[
            pltpu.VMEM((_BPW,), jnp.int32),
            pltpu.VMEM((_BPW,), jnp.int32),
            pltpu.VMEM((_C * 8, _D), jnp.float32),
            pltpu.VMEM((_C * 8, _D), jnp.float32),
            pltpu.VMEM((_BPW,), jnp.float32),
            pltpu.SemaphoreType.DMA,
            pltpu.SemaphoreType.DMA,
        ],
    )(u2, i2, user_emb, item_emb)


def kernel(u, i, user_emb, item_emb):
    u2 = u.astype(jnp.int32).reshape(_NW, _BPW)
    i2 = i.astype(jnp.int32).reshape(_NW, _BPW)
    return _pmf(u2, i2, user_emb, item_emb)
